# baseline (speedup 1.0000x reference)
import functools

import jax
import jax.numpy as jnp
from jax import lax
from jax.experimental import pallas as pl
from jax.experimental.pallas import tpu as pltpu
from jax.experimental.pallas import tpu_sc as plsc

B = 128
N = 32768
L = 16
NCH = N // L
NC, NS = 2, 16
NW = NC * NS
RPW = B // NW
TOPK = 10
SCH = 16
SEG = NCH // SCH
SPS = 8
NSUP = SEG // SPS
RB = 8

OFF_SVAL = 0
OFF_SIDX = 160
OFF_TVAL = 320
OFF_TIDX = 480
REC = 640


def _merge_chain(va, ia, vb, ib):
    c = (vb > va) | ((vb == va) & (ib < ia))
    return jnp.where(c, vb, va), jnp.where(c, ib, ia)


def _pass1(buf, seg_val, seg_idx, lane_i):
    ninf = jnp.full((L,), -jnp.inf, jnp.float32)
    zi = jnp.zeros((L,), jnp.int32)
    jconst = [jnp.full((L,), j * L, jnp.int32) for j in range(SCH)]

    def body(sg, _):
        base = sg * (SCH * L)
        sma, smia, smb, smib = ninf, zi, ninf, zi
        for j in range(SCH):
            x = buf[pl.ds(base + j * L, L)]
            if j % 2 == 0:
                c = x > sma
                sma = jnp.where(c, x, sma)
                smia = jnp.where(c, jconst[j], smia)
            else:
                c = x > smb
                smb = jnp.where(c, x, smb)
                smib = jnp.where(c, jconst[j], smib)
        sm, smi = _merge_chain(sma, smia, smb, smib)
        seg_val[pl.ds(sg * L, L)] = sm
        seg_idx[pl.ds(sg * L, L)] = smi + (base + lane_i)
        return 0

    lax.fori_loop(0, SEG, body, 0)


def _build_supseg(seg_val, supseg_val):
    def body(t, _):
        vs = [seg_val[pl.ds(t * (SPS * L) + j * L, L)] for j in range(SPS)]
        while len(vs) > 1:
            vs = [jnp.maximum(vs[i], vs[i + 1]) for i in range(0, len(vs), 2)]
        supseg_val[pl.ds(t * L, L)] = vs[0]
        return 0

    lax.fori_loop(0, NSUP, body, 0)


def _extract10(buf, seg_val, seg_idx, supseg_val, lane_i, stage,
               val_off, idx_off):
    ninf = jnp.full((L,), -jnp.inf, jnp.float32)
    zi = jnp.zeros((L,), jnp.int32)

    def body(k, _):
        bva, bta, bvb, btb = ninf, zi, ninf, zi
        for t in range(NSUP):
            v = supseg_val[pl.ds(t * L, L)]
            if t % 2 == 0:
                c = v > bva
                bva = jnp.where(c, v, bva)
                bta = jnp.where(c, zi + t, bta)
            else:
                c = v > bvb
                bvb = jnp.where(c, v, bvb)
                btb = jnp.where(c, zi + t, btb)
        bv, bt = _merge_chain(bva, bta, bvb, btb)
        dva, bsa, dvb, bsb = ninf, zi, ninf, zi
        for j in range(SPS):
            sj = bt * SPS + j
            g = plsc.load_gather(seg_val, [sj * L + lane_i])
            if j % 2 == 0:
                c = g > dva
                dva = jnp.where(c, g, dva)
                bsa = jnp.where(c, sj, bsa)
            else:
                c = g > dvb
                dvb = jnp.where(c, g, dvb)
                bsb = jnp.where(c, sj, bsb)
        _, bs = _merge_chain(dva, bsa, dvb, bsb)
        bi = plsc.load_gather(seg_idx, [bs * L + lane_i])
        plsc.store_scatter(buf, [bi], ninf)
        stage[pl.ds(val_off + k * L, L)] = bv
        stage[pl.ds(idx_off + k * L, L)] = bi.astype(jnp.float32)
        sbase = bs * (SCH * L) + lane_i
        nva, nia, nvb, nib = ninf, zi, ninf, zi
        for j in range(SCH):
            gidx = sbase + j * L
            g = plsc.load_gather(buf, [gidx])
            if j % 2 == 0:
                c = g > nva
                nva = jnp.where(c, g, nva)
                nia = jnp.where(c, gidx, nia)
            else:
                c = g > nvb
                nvb = jnp.where(c, g, nvb)
                nib = jnp.where(c, gidx, nib)
        nv, nvi = _merge_chain(nva, nia, nvb, nib)
        plsc.store_scatter(seg_val, [bs * L + lane_i], nv)
        plsc.store_scatter(seg_idx, [bs * L + lane_i], nvi)
        gs = [plsc.load_gather(seg_val, [(bt * SPS + j) * L + lane_i])
              for j in range(SPS)]
        while len(gs) > 1:
            gs = [jnp.maximum(gs[i], gs[i + 1]) for i in range(0, len(gs), 2)]
        plsc.store_scatter(supseg_val, [bt * L + lane_i], gs[0])
        return 0

    lax.fori_loop(0, TOPK, body, 0)


def _topk_row(buf, seg_val, seg_idx, supseg_val, lane_i, stage, voff, ioff):
    _pass1(buf, seg_val, seg_idx, lane_i)
    _build_supseg(seg_val, supseg_val)
    _extract10(buf, seg_val, seg_idx, supseg_val, lane_i, stage, voff, ioff)


def _sc_body(sem_hbm, struct_hbm, out_hbm,
             sem_a, sem_b, struct_v, seg_val, seg_idx, supseg_val,
             stage0, stage1, ds_sem_a, ds_sem_b, ds_str, ds_out):
    wid = lax.axis_index("s") * NC + lax.axis_index("c")
    lane_i = lax.broadcasted_iota(jnp.int32, (L,), 0)
    r0 = wid * RPW

    pltpu.async_copy(sem_hbm.at[r0], sem_a, ds_sem_a)

    def pair(it, _):
        ra = r0 + it * 2
        rb = ra + 1
        cp_str_a = pltpu.async_copy(struct_hbm.at[ra], struct_v, ds_str)
        cp_sem_b = pltpu.async_copy(sem_hbm.at[rb], sem_b, ds_sem_b)
        pltpu.make_async_copy(sem_hbm.at[ra], sem_a, ds_sem_a).wait()

        _topk_row(sem_a, seg_val, seg_idx, supseg_val, lane_i, stage0,
                  OFF_SVAL, OFF_SIDX)
        cp_str_a.wait()
        _topk_row(struct_v, seg_val, seg_idx, supseg_val, lane_i, stage0,
                  OFF_TVAL, OFF_TIDX)
        cp_out_a = pltpu.async_copy(stage0, out_hbm.at[ra], ds_out)

        cp_str_b = pltpu.async_copy(struct_hbm.at[rb], struct_v, ds_str)
        rn = jnp.minimum(ra + 2, r0 + RPW - 1)
        pltpu.async_copy(sem_hbm.at[rn], sem_a, ds_sem_a)
        cp_sem_b.wait()

        _topk_row(sem_b, seg_val, seg_idx, supseg_val, lane_i, stage1,
                  OFF_SVAL, OFF_SIDX)
        cp_str_b.wait()
        _topk_row(struct_v, seg_val, seg_idx, supseg_val, lane_i, stage1,
                  OFF_TVAL, OFF_TIDX)
        cp_out_b = pltpu.async_copy(stage1, out_hbm.at[rb], ds_out)
        cp_out_a.wait()
        cp_out_b.wait()
        return 0

    lax.fori_loop(0, RPW // 2, pair, 0)
    pltpu.make_async_copy(sem_hbm.at[r0 + RPW - 1], sem_a, ds_sem_a).wait()


def _sc_stage(s_sem, s_struct):
    mesh = plsc.VectorSubcoreMesh(core_axis_name="c", subcore_axis_name="s",
                                  num_cores=NC, num_subcores=NS)
    f32 = jnp.float32
    scratch = [
        pltpu.VMEM((N,), f32),
        pltpu.VMEM((N,), f32),
        pltpu.VMEM((N,), f32),
        pltpu.VMEM((SEG * L,), f32),
        pltpu.VMEM((SEG * L,), jnp.int32),
        pltpu.VMEM((NSUP * L,), f32),
        pltpu.VMEM((REC,), f32),
        pltpu.VMEM((REC,), f32),
        pltpu.SemaphoreType.DMA,
        pltpu.SemaphoreType.DMA,
        pltpu.SemaphoreType.DMA,
        pltpu.SemaphoreType.DMA,
    ]
    fn = pl.kernel(_sc_body,
                   out_type=[jax.ShapeDtypeStruct((B, REC), f32)],
                   mesh=mesh,
                   compiler_params=pltpu.CompilerParams(
                       needs_layout_passes=False),
                   scratch_types=scratch)
    return fn(s_sem, s_struct)


def _stats_body(x_ref, out_ref):
    nf = jnp.float32(N)
    x = x_ref[...]
    m = jnp.max(x, axis=1, keepdims=True)
    mean = jnp.sum(x, axis=1, keepdims=True) / nf
    var = jnp.sum(x * x, axis=1, keepdims=True) / nf - mean * mean
    std = jnp.sqrt(jnp.maximum(var, 0.0))
    e = jnp.exp(x - m)
    s1 = jnp.sum(e, axis=1, keepdims=True)
    s2 = jnp.sum(e * x, axis=1, keepdims=True)
    ent = m + jnp.log(s1) - s2 / s1
    gap = m - mean
    z = jnp.zeros_like(mean)
    out_ref[...] = jnp.concatenate(
        [mean, std, m, gap, ent, z, z, z], axis=1)


def _stats_stage(s_sem):
    return pl.pallas_call(
        _stats_body,
        grid=(B // RB,),
        in_specs=[pl.BlockSpec((RB, N), lambda i: (i, 0))],
        out_specs=pl.BlockSpec((RB, 8), lambda i: (i, 0)),
        out_shape=jax.ShapeDtypeStruct((B, 8), jnp.float32),
    )(s_sem)


def _tc_body(stats_ref, comb_ref, relid_ref, dirid_ref, rel_emb_ref,
             dir_emb_ref, w1_ref, b1_ref, w2_ref, b2_ref, inv_ref, out_ref):
    comb = comb_ref[...]
    stats = stats_ref[...]

    def select10(vals, idx):
        v = vals
        sels = []
        for _ in range(TOPK):
            mx = jnp.max(v, axis=1, keepdims=True)
            ci = jnp.where(v == mx, idx, jnp.float32(1e9))
            si = jnp.min(ci, axis=1, keepdims=True)
            sels.append(si)
            v = jnp.where(idx == si, -jnp.inf, v)
        return sels

    semsel = select10(comb[:, OFF_SVAL:OFF_SIDX], comb[:, OFF_SIDX:OFF_TVAL])
    strsel = jnp.concatenate(
        select10(comb[:, OFF_TVAL:OFF_TIDX], comb[:, OFF_TIDX:REC]), axis=1)

    match = jnp.zeros((B,), jnp.float32)
    for i in range(TOPK):
        hit = jnp.max((semsel[i] == strsel).astype(jnp.float32), axis=1)
        match = match + hit
    agree = match * inv_ref[0, 0]

    rel_oh = (relid_ref[...] ==
              lax.broadcasted_iota(jnp.int32, (B, rel_emb_ref.shape[0]), 1)
              ).astype(jnp.float32)
    dir_oh = (dirid_ref[...] ==
              lax.broadcasted_iota(jnp.int32, (B, 2), 1)).astype(jnp.float32)
    rel_vec = jnp.dot(rel_oh, rel_emb_ref[...],
                      preferred_element_type=jnp.float32)
    dir_vec = jnp.dot(dir_oh, dir_emb_ref[...],
                      preferred_element_type=jnp.float32)

    x = jnp.concatenate(
        [stats[:, 0:5], agree[:, None], rel_vec, dir_vec], axis=1)
    h = jnp.maximum(jnp.dot(x, w1_ref[...],
                            preferred_element_type=jnp.float32) + b1_ref[...],
                    0.0)
    z = jnp.dot(h, w2_ref[...], preferred_element_type=jnp.float32) + b2_ref[...]
    r = 1.0 / (1.0 + jnp.exp(-z))
    out_ref[...] = jnp.clip(r, 0.05, 0.95)


def kernel(s_sem, s_struct, rel_ids, dir_ids, topm, rel_emb, dir_emb,
           W1, b1, W2, b2):
    (comb,) = _sc_stage(s_sem, s_struct)
    stats = _stats_stage(s_sem)

    inv_topm = (1.0 / jnp.asarray(topm, jnp.float32)).reshape(1, 1)
    relid = rel_ids.astype(jnp.int32).reshape(B, 1)
    dirid = dir_ids.astype(jnp.int32).reshape(B, 1)

    out = pl.pallas_call(
        _tc_body,
        out_shape=jax.ShapeDtypeStruct((B, 1), jnp.float32),
    )(stats, comb, relid, dirid, rel_emb, dir_emb,
      W1, b1.reshape(1, -1), W2, b2.reshape(1, 1), inv_topm)
    return out[:, 0]


# speedup vs baseline: 1.0106x; 1.0106x over previous
import functools

import jax
import jax.numpy as jnp
from jax import lax
from jax.experimental import pallas as pl
from jax.experimental.pallas import tpu as pltpu
from jax.experimental.pallas import tpu_sc as plsc

B = 128
N = 32768
L = 16
NCH = N // L
NC, NS = 2, 16
NW = NC * NS
RPW = B // NW
TOPK = 10
SCH = 16
SEG = NCH // SCH
SPS = 8
NSUP = SEG // SPS
RB = 8

OFF_SVAL = 0
OFF_SIDX = 160
OFF_TVAL = 320
OFF_TIDX = 480
REC = 640


def _merge_chain(va, ia, vb, ib):
    c = (vb > va) | ((vb == va) & (ib < ia))
    return jnp.where(c, vb, va), jnp.where(c, ib, ia)


def _pass1(buf, seg_val, seg_idx, lane_i):
    ninf = jnp.full((L,), -jnp.inf, jnp.float32)
    zi = jnp.zeros((L,), jnp.int32)
    jconst = [jnp.full((L,), j * L, jnp.int32) for j in range(SCH)]

    def one_seg(base):
        sma, smia, smb, smib = ninf, zi, ninf, zi
        for j in range(SCH):
            x = buf[pl.ds(base + j * L, L)]
            if j % 2 == 0:
                c = x > sma
                sma = jnp.where(c, x, sma)
                smia = jnp.where(c, jconst[j], smia)
            else:
                c = x > smb
                smb = jnp.where(c, x, smb)
                smib = jnp.where(c, jconst[j], smib)
        return _merge_chain(sma, smia, smb, smib)

    def body(g, _):
        for u in range(2):
            sg = g * 2 + u
            base = sg * (SCH * L)
            sm, smi = one_seg(base)
            seg_val[pl.ds(sg * L, L)] = sm
            seg_idx[pl.ds(sg * L, L)] = smi + (base + lane_i)
        return 0

    lax.fori_loop(0, SEG // 2, body, 0)


def _build_supseg(seg_val, supseg_val):
    def body(t, _):
        vs = [seg_val[pl.ds(t * (SPS * L) + j * L, L)] for j in range(SPS)]
        while len(vs) > 1:
            vs = [jnp.maximum(vs[i], vs[i + 1]) for i in range(0, len(vs), 2)]
        supseg_val[pl.ds(t * L, L)] = vs[0]
        return 0

    lax.fori_loop(0, NSUP, body, 0)


def _extract10(buf, seg_val, seg_idx, supseg_val, lane_i, stage,
               val_off, idx_off):
    ninf = jnp.full((L,), -jnp.inf, jnp.float32)
    zi = jnp.zeros((L,), jnp.int32)

    def body(k, _):
        bva, bta, bvb, btb = ninf, zi, ninf, zi
        for t in range(NSUP):
            v = supseg_val[pl.ds(t * L, L)]
            if t % 2 == 0:
                c = v > bva
                bva = jnp.where(c, v, bva)
                bta = jnp.where(c, zi + t, bta)
            else:
                c = v > bvb
                bvb = jnp.where(c, v, bvb)
                btb = jnp.where(c, zi + t, btb)
        bv, bt = _merge_chain(bva, bta, bvb, btb)
        dva, bsa, dvb, bsb = ninf, zi, ninf, zi
        for j in range(SPS):
            sj = bt * SPS + j
            g = plsc.load_gather(seg_val, [sj * L + lane_i])
            if j % 2 == 0:
                c = g > dva
                dva = jnp.where(c, g, dva)
                bsa = jnp.where(c, sj, bsa)
            else:
                c = g > dvb
                dvb = jnp.where(c, g, dvb)
                bsb = jnp.where(c, sj, bsb)
        _, bs = _merge_chain(dva, bsa, dvb, bsb)
        bi = plsc.load_gather(seg_idx, [bs * L + lane_i])
        plsc.store_scatter(buf, [bi], ninf)
        stage[pl.ds(val_off + k * L, L)] = bv
        stage[pl.ds(idx_off + k * L, L)] = bi.astype(jnp.float32)
        sbase = bs * (SCH * L) + lane_i
        nva, nia, nvb, nib = ninf, zi, ninf, zi
        for j in range(SCH):
            gidx = sbase + j * L
            g = plsc.load_gather(buf, [gidx])
            if j % 2 == 0:
                c = g > nva
                nva = jnp.where(c, g, nva)
                nia = jnp.where(c, gidx, nia)
            else:
                c = g > nvb
                nvb = jnp.where(c, g, nvb)
                nib = jnp.where(c, gidx, nib)
        nv, nvi = _merge_chain(nva, nia, nvb, nib)
        plsc.store_scatter(seg_val, [bs * L + lane_i], nv)
        plsc.store_scatter(seg_idx, [bs * L + lane_i], nvi)
        gs = [plsc.load_gather(seg_val, [(bt * SPS + j) * L + lane_i])
              for j in range(SPS)]
        while len(gs) > 1:
            gs = [jnp.maximum(gs[i], gs[i + 1]) for i in range(0, len(gs), 2)]
        plsc.store_scatter(supseg_val, [bt * L + lane_i], gs[0])
        return 0

    lax.fori_loop(0, TOPK, body, 0)


def _topk_row(buf, seg_val, seg_idx, supseg_val, lane_i, stage, voff, ioff):
    _pass1(buf, seg_val, seg_idx, lane_i)
    _build_supseg(seg_val, supseg_val)
    _extract10(buf, seg_val, seg_idx, supseg_val, lane_i, stage, voff, ioff)


def _sc_body(sem_hbm, struct_hbm, out_hbm,
             sem_a, sem_b, struct_v, seg_val, seg_idx, supseg_val,
             stage0, stage1, stage2, stage3, ds_sem, ds_str, ds_out):
    wid = lax.axis_index("s") * NC + lax.axis_index("c")
    lane_i = lax.broadcasted_iota(jnp.int32, (L,), 0)
    r0 = wid * RPW

    sem_bufs = [sem_a, sem_b]
    stage_bufs = [stage0, stage1, stage2, stage3]
    cp_sem = pltpu.async_copy(sem_hbm.at[r0], sem_a, ds_sem)
    out_cps = []
    for rr in range(RPW):
        r = r0 + rr
        cur = sem_bufs[rr % 2]
        stage = stage_bufs[rr]
        cp_struct = pltpu.async_copy(struct_hbm.at[r], struct_v, ds_str)
        cp_sem.wait()
        if rr + 1 < RPW:
            cp_sem = pltpu.async_copy(sem_hbm.at[r + 1],
                                      sem_bufs[(rr + 1) % 2], ds_sem)

        _topk_row(cur, seg_val, seg_idx, supseg_val, lane_i, stage,
                  OFF_SVAL, OFF_SIDX)
        cp_struct.wait()
        _topk_row(struct_v, seg_val, seg_idx, supseg_val, lane_i, stage,
                  OFF_TVAL, OFF_TIDX)

        out_cps.append(pltpu.async_copy(stage, out_hbm.at[r], ds_out))
    for cp in out_cps:
        cp.wait()


def _sc_stage(s_sem, s_struct):
    mesh = plsc.VectorSubcoreMesh(core_axis_name="c", subcore_axis_name="s",
                                  num_cores=NC, num_subcores=NS)
    f32 = jnp.float32
    scratch = [
        pltpu.VMEM((N,), f32),
        pltpu.VMEM((N,), f32),
        pltpu.VMEM((N,), f32),
        pltpu.VMEM((SEG * L,), f32),
        pltpu.VMEM((SEG * L,), jnp.int32),
        pltpu.VMEM((NSUP * L,), f32),
        pltpu.VMEM((REC,), f32),
        pltpu.VMEM((REC,), f32),
        pltpu.VMEM((REC,), f32),
        pltpu.VMEM((REC,), f32),
        pltpu.SemaphoreType.DMA,
        pltpu.SemaphoreType.DMA,
        pltpu.SemaphoreType.DMA,
    ]
    fn = pl.kernel(_sc_body,
                   out_type=[jax.ShapeDtypeStruct((B, REC), f32)],
                   mesh=mesh,
                   compiler_params=pltpu.CompilerParams(
                       needs_layout_passes=False),
                   scratch_types=scratch)
    return fn(s_sem, s_struct)


def _stats_body(x_ref, out_ref):
    nf = jnp.float32(N)
    x = x_ref[...]
    m = jnp.max(x, axis=1, keepdims=True)
    mean = jnp.sum(x, axis=1, keepdims=True) / nf
    var = jnp.sum(x * x, axis=1, keepdims=True) / nf - mean * mean
    std = jnp.sqrt(jnp.maximum(var, 0.0))
    e = jnp.exp(x - m)
    s1 = jnp.sum(e, axis=1, keepdims=True)
    s2 = jnp.sum(e * x, axis=1, keepdims=True)
    ent = m + jnp.log(s1) - s2 / s1
    gap = m - mean
    z = jnp.zeros_like(mean)
    out_ref[...] = jnp.concatenate(
        [mean, std, m, gap, ent, z, z, z], axis=1)


def _stats_stage(s_sem):
    return pl.pallas_call(
        _stats_body,
        grid=(B // RB,),
        in_specs=[pl.BlockSpec((RB, N), lambda i: (i, 0))],
        out_specs=pl.BlockSpec((RB, 8), lambda i: (i, 0)),
        out_shape=jax.ShapeDtypeStruct((B, 8), jnp.float32),
    )(s_sem)


def _tc_body(stats_ref, comb_ref, relid_ref, dirid_ref, rel_emb_ref,
             dir_emb_ref, w1_ref, b1_ref, w2_ref, b2_ref, inv_ref, out_ref):
    comb = comb_ref[...]
    stats = stats_ref[...]

    def select10(vals, idx):
        v = vals
        sels = []
        for _ in range(TOPK):
            mx = jnp.max(v, axis=1, keepdims=True)
            ci = jnp.where(v == mx, idx, jnp.float32(1e9))
            si = jnp.min(ci, axis=1, keepdims=True)
            sels.append(si)
            v = jnp.where(idx == si, -jnp.inf, v)
        return sels

    semsel = select10(comb[:, OFF_SVAL:OFF_SIDX], comb[:, OFF_SIDX:OFF_TVAL])
    strsel = jnp.concatenate(
        select10(comb[:, OFF_TVAL:OFF_TIDX], comb[:, OFF_TIDX:REC]), axis=1)

    match = jnp.zeros((B,), jnp.float32)
    for i in range(TOPK):
        hit = jnp.max((semsel[i] == strsel).astype(jnp.float32), axis=1)
        match = match + hit
    agree = match * inv_ref[0, 0]

    rel_oh = (relid_ref[...] ==
              lax.broadcasted_iota(jnp.int32, (B, rel_emb_ref.shape[0]), 1)
              ).astype(jnp.float32)
    dir_oh = (dirid_ref[...] ==
              lax.broadcasted_iota(jnp.int32, (B, 2), 1)).astype(jnp.float32)
    rel_vec = jnp.dot(rel_oh, rel_emb_ref[...],
                      preferred_element_type=jnp.float32)
    dir_vec = jnp.dot(dir_oh, dir_emb_ref[...],
                      preferred_element_type=jnp.float32)

    x = jnp.concatenate(
        [stats[:, 0:5], agree[:, None], rel_vec, dir_vec], axis=1)
    h = jnp.maximum(jnp.dot(x, w1_ref[...],
                            preferred_element_type=jnp.float32) + b1_ref[...],
                    0.0)
    z = jnp.dot(h, w2_ref[...], preferred_element_type=jnp.float32) + b2_ref[...]
    r = 1.0 / (1.0 + jnp.exp(-z))
    out_ref[...] = jnp.clip(r, 0.05, 0.95)


def kernel(s_sem, s_struct, rel_ids, dir_ids, topm, rel_emb, dir_emb,
           W1, b1, W2, b2):
    (comb,) = _sc_stage(s_sem, s_struct)
    stats = _stats_stage(s_sem)

    inv_topm = (1.0 / jnp.asarray(topm, jnp.float32)).reshape(1, 1)
    relid = rel_ids.astype(jnp.int32).reshape(B, 1)
    dirid = dir_ids.astype(jnp.int32).reshape(B, 1)

    out = pl.pallas_call(
        _tc_body,
        out_shape=jax.ShapeDtypeStruct((B, 1), jnp.float32),
    )(stats, comb, relid, dirid, rel_emb, dir_emb,
      W1, b1.reshape(1, -1), W2, b2.reshape(1, 1), inv_topm)
    return out[:, 0]


# speedup vs baseline: 1.0121x; 1.0014x over previous
import functools

import jax
import jax.numpy as jnp
from jax import lax
from jax.experimental import pallas as pl
from jax.experimental.pallas import tpu as pltpu
from jax.experimental.pallas import tpu_sc as plsc

B = 128
N = 32768
L = 16
NCH = N // L
NC, NS = 2, 16
NW = NC * NS
RPW = B // NW
TOPK = 10
SCH = 16
SEG = NCH // SCH
SPS = 8
NSUP = SEG // SPS
RB = 8

OFF_SVAL = 0
OFF_SIDX = 160
OFF_TVAL = 320
OFF_TIDX = 480
REC = 640


def _merge_chain(va, ia, vb, ib):
    c = (vb > va) | ((vb == va) & (ib < ia))
    return jnp.where(c, vb, va), jnp.where(c, ib, ia)


def _pass1(buf, seg_val, seg_idx, lane_i):
    ninf = jnp.full((L,), -jnp.inf, jnp.float32)
    zi = jnp.zeros((L,), jnp.int32)
    jconst = [jnp.full((L,), j * L, jnp.int32) for j in range(SCH)]

    def one_seg(base):
        sma, smia, smb, smib = ninf, zi, ninf, zi
        for j in range(SCH):
            x = buf[pl.ds(base + j * L, L)]
            if j % 2 == 0:
                c = x > sma
                sma = jnp.where(c, x, sma)
                smia = jnp.where(c, jconst[j], smia)
            else:
                c = x > smb
                smb = jnp.where(c, x, smb)
                smib = jnp.where(c, jconst[j], smib)
        return _merge_chain(sma, smia, smb, smib)

    def body(g, _):
        for u in range(2):
            sg = g * 2 + u
            base = sg * (SCH * L)
            sm, smi = one_seg(base)
            seg_val[pl.ds(sg * L, L)] = sm
            seg_idx[pl.ds(sg * L, L)] = smi + (base + lane_i)
        return 0

    lax.fori_loop(0, SEG // 2, body, 0)


def _build_supseg(seg_val, supseg_val):
    def body(t, _):
        vs = [seg_val[pl.ds(t * (SPS * L) + j * L, L)] for j in range(SPS)]
        while len(vs) > 1:
            vs = [jnp.maximum(vs[i], vs[i + 1]) for i in range(0, len(vs), 2)]
        supseg_val[pl.ds(t * L, L)] = vs[0]
        return 0

    lax.fori_loop(0, NSUP, body, 0)


def _extract10(buf, seg_val, seg_idx, supseg_val, lane_i, stage,
               val_off, idx_off):
    ninf = jnp.full((L,), -jnp.inf, jnp.float32)
    zi = jnp.zeros((L,), jnp.int32)

    def body(k, _):
        bva, bta, bvb, btb = ninf, zi, ninf, zi
        for t in range(NSUP):
            v = supseg_val[pl.ds(t * L, L)]
            if t % 2 == 0:
                c = v > bva
                bva = jnp.where(c, v, bva)
                bta = jnp.where(c, zi + t, bta)
            else:
                c = v > bvb
                bvb = jnp.where(c, v, bvb)
                btb = jnp.where(c, zi + t, btb)
        bv, bt = _merge_chain(bva, bta, bvb, btb)
        dva, bsa, dvb, bsb = ninf, zi, ninf, zi
        for j in range(SPS):
            sj = bt * SPS + j
            g = plsc.load_gather(seg_val, [sj * L + lane_i])
            if j % 2 == 0:
                c = g > dva
                dva = jnp.where(c, g, dva)
                bsa = jnp.where(c, sj, bsa)
            else:
                c = g > dvb
                dvb = jnp.where(c, g, dvb)
                bsb = jnp.where(c, sj, bsb)
        _, bs = _merge_chain(dva, bsa, dvb, bsb)
        bi = plsc.load_gather(seg_idx, [bs * L + lane_i])
        plsc.store_scatter(buf, [bi], ninf)
        stage[pl.ds(val_off + k * L, L)] = bv
        stage[pl.ds(idx_off + k * L, L)] = bi.astype(jnp.float32)
        sbase = bs * (SCH * L) + lane_i
        nva, nia, nvb, nib = ninf, zi, ninf, zi
        for j in range(SCH):
            gidx = sbase + j * L
            g = plsc.load_gather(buf, [gidx])
            if j % 2 == 0:
                c = g > nva
                nva = jnp.where(c, g, nva)
                nia = jnp.where(c, gidx, nia)
            else:
                c = g > nvb
                nvb = jnp.where(c, g, nvb)
                nib = jnp.where(c, gidx, nib)
        nv, nvi = _merge_chain(nva, nia, nvb, nib)
        plsc.store_scatter(seg_val, [bs * L + lane_i], nv)
        plsc.store_scatter(seg_idx, [bs * L + lane_i], nvi)
        gs = [plsc.load_gather(seg_val, [(bt * SPS + j) * L + lane_i])
              for j in range(SPS)]
        while len(gs) > 1:
            gs = [jnp.maximum(gs[i], gs[i + 1]) for i in range(0, len(gs), 2)]
        plsc.store_scatter(supseg_val, [bt * L + lane_i], gs[0])
        return 0

    lax.fori_loop(0, TOPK, body, 0)


def _topk_row(buf, seg_val, seg_idx, supseg_val, lane_i, stage, voff, ioff):
    _pass1(buf, seg_val, seg_idx, lane_i)
    _build_supseg(seg_val, supseg_val)
    _extract10(buf, seg_val, seg_idx, supseg_val, lane_i, stage, voff, ioff)


def _sc_body(sem_hbm, struct_hbm, out_hbm,
             sem_a, sem_b, struct_v, seg_val, seg_idx, supseg_val,
             stage0, stage1, stage2, stage3, ds_sem, ds_str, ds_out):
    wid = lax.axis_index("s") * NC + lax.axis_index("c")
    lane_i = lax.broadcasted_iota(jnp.int32, (L,), 0)
    r0 = wid * RPW

    sem_bufs = [sem_a, sem_b]
    stage_bufs = [stage0, stage1, stage2, stage3]
    cp_sem = pltpu.async_copy(sem_hbm.at[r0], sem_a, ds_sem)
    out_cps = []
    for rr in range(RPW):
        r = r0 + rr
        cur = sem_bufs[rr % 2]
        stage = stage_bufs[rr]
        cp_struct = pltpu.async_copy(struct_hbm.at[r], struct_v, ds_str)
        cp_sem.wait()
        if rr + 1 < RPW:
            cp_sem = pltpu.async_copy(sem_hbm.at[r + 1],
                                      sem_bufs[(rr + 1) % 2], ds_sem)

        _topk_row(cur, seg_val, seg_idx, supseg_val, lane_i, stage,
                  OFF_SVAL, OFF_SIDX)
        cp_struct.wait()
        _topk_row(struct_v, seg_val, seg_idx, supseg_val, lane_i, stage,
                  OFF_TVAL, OFF_TIDX)

        out_cps.append(pltpu.async_copy(stage, out_hbm.at[r], ds_out))
    for cp in out_cps:
        cp.wait()


def _sc_stage(s_sem, s_struct):
    mesh = plsc.VectorSubcoreMesh(core_axis_name="c", subcore_axis_name="s",
                                  num_cores=NC, num_subcores=NS)
    f32 = jnp.float32
    scratch = [
        pltpu.VMEM((N,), f32),
        pltpu.VMEM((N,), f32),
        pltpu.VMEM((N,), f32),
        pltpu.VMEM((SEG * L,), f32),
        pltpu.VMEM((SEG * L,), jnp.int32),
        pltpu.VMEM((NSUP * L,), f32),
        pltpu.VMEM((REC,), f32),
        pltpu.VMEM((REC,), f32),
        pltpu.VMEM((REC,), f32),
        pltpu.VMEM((REC,), f32),
        pltpu.SemaphoreType.DMA,
        pltpu.SemaphoreType.DMA,
        pltpu.SemaphoreType.DMA,
    ]
    fn = pl.kernel(_sc_body,
                   out_type=[jax.ShapeDtypeStruct((B, REC), f32)],
                   mesh=mesh,
                   compiler_params=pltpu.CompilerParams(
                       needs_layout_passes=False),
                   scratch_types=scratch)
    return fn(s_sem, s_struct)


def _stats_body(x_ref, relid_ref, dirid_ref, rel_emb_ref, dir_emb_ref,
                w1_ref, b1_ref, out_ref, pre_ref):
    nf = jnp.float32(N)
    x = x_ref[...]
    m = jnp.max(x, axis=1, keepdims=True)
    mean = jnp.sum(x, axis=1, keepdims=True) / nf
    var = jnp.sum(x * x, axis=1, keepdims=True) / nf - mean * mean
    std = jnp.sqrt(jnp.maximum(var, 0.0))
    e = jnp.exp(x - m)
    s1 = jnp.sum(e, axis=1, keepdims=True)
    s2 = jnp.sum(e * x, axis=1, keepdims=True)
    ent = m + jnp.log(s1) - s2 / s1
    gap = m - mean
    z = jnp.zeros_like(mean)
    out_ref[...] = jnp.concatenate(
        [mean, std, m, gap, ent, z, z, z], axis=1)

    rel_oh = (relid_ref[...] ==
              lax.broadcasted_iota(jnp.int32, (RB, rel_emb_ref.shape[0]), 1)
              ).astype(jnp.float32)
    dir_oh = (dirid_ref[...] ==
              lax.broadcasted_iota(jnp.int32, (RB, 2), 1)).astype(jnp.float32)
    rel_vec = jnp.dot(rel_oh, rel_emb_ref[...],
                      preferred_element_type=jnp.float32)
    dir_vec = jnp.dot(dir_oh, dir_emb_ref[...],
                      preferred_element_type=jnp.float32)
    pre_ref[...] = (jnp.dot(rel_vec, w1_ref[6:38, :],
                            preferred_element_type=jnp.float32) +
                    jnp.dot(dir_vec, w1_ref[38:46, :],
                            preferred_element_type=jnp.float32) +
                    b1_ref[...])


def _stats_stage(s_sem, relid, dirid, rel_emb, dir_emb, W1, b1):
    hid = W1.shape[1]
    return pl.pallas_call(
        _stats_body,
        grid=(B // RB,),
        in_specs=[
            pl.BlockSpec((RB, N), lambda i: (i, 0)),
            pl.BlockSpec((RB, 1), lambda i: (i, 0)),
            pl.BlockSpec((RB, 1), lambda i: (i, 0)),
            pl.BlockSpec(rel_emb.shape, lambda i: (0, 0)),
            pl.BlockSpec(dir_emb.shape, lambda i: (0, 0)),
            pl.BlockSpec(W1.shape, lambda i: (0, 0)),
            pl.BlockSpec((1, hid), lambda i: (0, 0)),
        ],
        out_specs=[
            pl.BlockSpec((RB, 8), lambda i: (i, 0)),
            pl.BlockSpec((RB, hid), lambda i: (i, 0)),
        ],
        out_shape=[
            jax.ShapeDtypeStruct((B, 8), jnp.float32),
            jax.ShapeDtypeStruct((B, hid), jnp.float32),
        ],
    )(s_sem, relid, dirid, rel_emb, dir_emb, W1, b1.reshape(1, hid))


def _tc_body(stats_ref, comb_ref, pre_ref, w1_ref, w2_ref, b2_ref, inv_ref,
             out_ref):
    comb = comb_ref[...]
    stats = stats_ref[...]

    vA = comb[:, OFF_SVAL:OFF_SIDX]
    iA = comb[:, OFF_SIDX:OFF_TVAL]
    vB = comb[:, OFF_TVAL:OFF_TIDX]
    iB = comb[:, OFF_TIDX:REC]
    selA, selB = [], []
    for _ in range(TOPK):
        mxA = jnp.max(vA, axis=1, keepdims=True)
        mxB = jnp.max(vB, axis=1, keepdims=True)
        siA = jnp.min(jnp.where(vA == mxA, iA, jnp.float32(1e9)),
                      axis=1, keepdims=True)
        siB = jnp.min(jnp.where(vB == mxB, iB, jnp.float32(1e9)),
                      axis=1, keepdims=True)
        selA.append(siA)
        selB.append(siB)
        vA = jnp.where(iA == siA, -jnp.inf, vA)
        vB = jnp.where(iB == siB, -jnp.inf, vB)
    strsel = jnp.concatenate(selB, axis=1)

    match = jnp.zeros((B,), jnp.float32)
    for i in range(TOPK):
        hit = jnp.max((selA[i] == strsel).astype(jnp.float32), axis=1)
        match = match + hit
    agree = match * inv_ref[0, 0]

    x = jnp.concatenate([stats[:, 0:5], agree[:, None]], axis=1)
    h = jnp.maximum(jnp.dot(x, w1_ref[0:6, :],
                            preferred_element_type=jnp.float32) + pre_ref[...],
                    0.0)
    z = jnp.dot(h, w2_ref[...], preferred_element_type=jnp.float32) + b2_ref[...]
    r = 1.0 / (1.0 + jnp.exp(-z))
    out_ref[...] = jnp.clip(r, 0.05, 0.95)


def kernel(s_sem, s_struct, rel_ids, dir_ids, topm, rel_emb, dir_emb,
           W1, b1, W2, b2):
    (comb,) = _sc_stage(s_sem, s_struct)

    inv_topm = (1.0 / jnp.asarray(topm, jnp.float32)).reshape(1, 1)
    relid = rel_ids.astype(jnp.int32).reshape(B, 1)
    dirid = dir_ids.astype(jnp.int32).reshape(B, 1)
    stats, pre = _stats_stage(s_sem, relid, dirid, rel_emb, dir_emb, W1, b1)

    out = pl.pallas_call(
        _tc_body,
        out_shape=jax.ShapeDtypeStruct((B, 1), jnp.float32),
    )(stats, comb, pre, W1, W2, b2.reshape(1, 1), inv_topm)
    return out[:, 0]


# speedup vs baseline: 1.0452x; 1.0327x over previous
import functools

import jax
import jax.numpy as jnp
from jax import lax
from jax.experimental import pallas as pl
from jax.experimental.pallas import tpu as pltpu
from jax.experimental.pallas import tpu_sc as plsc

B = 128
N = 32768
L = 16
NCH = N // L
NC, NS = 2, 16
NW = NC * NS
RPW = B // NW
TOPK = 10
SCH = 16
SEG = NCH // SCH
SPS = 8
NSUP = SEG // SPS
RB = 8

OFF_SVAL = 0
OFF_SIDX = 160
OFF_TVAL = 320
OFF_TIDX = 480
REC = 640


def _merge_chain(va, ia, vb, ib):
    c = (vb > va) | ((vb == va) & (ib < ia))
    return jnp.where(c, vb, va), jnp.where(c, ib, ia)


def _pass1(buf, seg_val):
    ninf = jnp.full((L,), -jnp.inf, jnp.float32)

    def one_seg(base):
        sa, sb = ninf, ninf
        for j in range(SCH):
            x = buf[pl.ds(base + j * L, L)]
            if j % 2 == 0:
                sa = jnp.maximum(sa, x)
            else:
                sb = jnp.maximum(sb, x)
        return jnp.maximum(sa, sb)

    def body(g, _):
        for u in range(2):
            sg = g * 2 + u
            seg_val[pl.ds(sg * L, L)] = one_seg(sg * (SCH * L))
        return 0

    lax.fori_loop(0, SEG // 2, body, 0)


def _build_supseg(seg_val, supseg_val):
    def body(t, _):
        vs = [seg_val[pl.ds(t * (SPS * L) + j * L, L)] for j in range(SPS)]
        while len(vs) > 1:
            vs = [jnp.maximum(vs[i], vs[i + 1]) for i in range(0, len(vs), 2)]
        supseg_val[pl.ds(t * L, L)] = vs[0]
        return 0

    lax.fori_loop(0, NSUP, body, 0)


def _extract10(buf, seg_val, supseg_val, lane_i, stage, val_off, idx_off):
    ninf = jnp.full((L,), -jnp.inf, jnp.float32)
    zi = jnp.zeros((L,), jnp.int32)

    def body(k, _):
        bva, bta, bvb, btb = ninf, zi, ninf, zi
        for t in range(NSUP):
            v = supseg_val[pl.ds(t * L, L)]
            if t % 2 == 0:
                c = v > bva
                bva = jnp.where(c, v, bva)
                bta = jnp.where(c, zi + t, bta)
            else:
                c = v > bvb
                bvb = jnp.where(c, v, bvb)
                btb = jnp.where(c, zi + t, btb)
        bv, bt = _merge_chain(bva, bta, bvb, btb)
        dva, bsa, dvb, bsb = ninf, zi, ninf, zi
        for j in range(SPS):
            sj = bt * SPS + j
            g = plsc.load_gather(seg_val, [sj * L + lane_i])
            if j % 2 == 0:
                c = g > dva
                dva = jnp.where(c, g, dva)
                bsa = jnp.where(c, sj, bsa)
            else:
                c = g > dvb
                dvb = jnp.where(c, g, dvb)
                bsb = jnp.where(c, sj, bsb)
        _, bs = _merge_chain(dva, bsa, dvb, bsb)
        sbase = bs * (SCH * L) + lane_i
        nva, nia, nvb, nib = ninf, zi, ninf, zi
        for j in range(SCH):
            gidx = sbase + j * L
            g = plsc.load_gather(buf, [gidx])
            if j % 2 == 0:
                c = g > nva
                nva = jnp.where(c, g, nva)
                nia = jnp.where(c, gidx, nia)
            else:
                c = g > nvb
                nvb = jnp.where(c, g, nvb)
                nib = jnp.where(c, gidx, nib)
        _, bi = _merge_chain(nva, nia, nvb, nib)
        plsc.store_scatter(buf, [bi], ninf)
        stage[pl.ds(val_off + k * L, L)] = bv
        stage[pl.ds(idx_off + k * L, L)] = bi.astype(jnp.float32)
        gs = [plsc.load_gather(buf, [sbase + j * L]) for j in range(SCH)]
        while len(gs) > 1:
            gs = [jnp.maximum(gs[i], gs[i + 1]) for i in range(0, len(gs), 2)]
        plsc.store_scatter(seg_val, [bs * L + lane_i], gs[0])
        hs = [plsc.load_gather(seg_val, [(bt * SPS + j) * L + lane_i])
              for j in range(SPS)]
        while len(hs) > 1:
            hs = [jnp.maximum(hs[i], hs[i + 1]) for i in range(0, len(hs), 2)]
        plsc.store_scatter(supseg_val, [bt * L + lane_i], hs[0])
        return 0

    lax.fori_loop(0, TOPK, body, 0)


def _topk_row(buf, seg_val, supseg_val, lane_i, stage, voff, ioff):
    _pass1(buf, seg_val)
    _build_supseg(seg_val, supseg_val)
    _extract10(buf, seg_val, supseg_val, lane_i, stage, voff, ioff)


def _sc_body(sem_hbm, struct_hbm, out_hbm,
             sem_a, sem_b, struct_v, seg_val, supseg_val,
             stage0, stage1, stage2, stage3, ds_sem, ds_str, ds_out):
    wid = lax.axis_index("s") * NC + lax.axis_index("c")
    lane_i = lax.broadcasted_iota(jnp.int32, (L,), 0)
    r0 = wid * RPW

    sem_bufs = [sem_a, sem_b]
    stage_bufs = [stage0, stage1, stage2, stage3]
    cp_sem = pltpu.async_copy(sem_hbm.at[r0], sem_a, ds_sem)
    out_cps = []
    for rr in range(RPW):
        r = r0 + rr
        cur = sem_bufs[rr % 2]
        stage = stage_bufs[rr]
        cp_struct = pltpu.async_copy(struct_hbm.at[r], struct_v, ds_str)
        cp_sem.wait()
        if rr + 1 < RPW:
            cp_sem = pltpu.async_copy(sem_hbm.at[r + 1],
                                      sem_bufs[(rr + 1) % 2], ds_sem)

        _topk_row(cur, seg_val, supseg_val, lane_i, stage,
                  OFF_SVAL, OFF_SIDX)
        cp_struct.wait()
        _topk_row(struct_v, seg_val, supseg_val, lane_i, stage,
                  OFF_TVAL, OFF_TIDX)

        out_cps.append(pltpu.async_copy(stage, out_hbm.at[r], ds_out))
    for cp in out_cps:
        cp.wait()


def _sc_stage(s_sem, s_struct):
    mesh = plsc.VectorSubcoreMesh(core_axis_name="c", subcore_axis_name="s",
                                  num_cores=NC, num_subcores=NS)
    f32 = jnp.float32
    scratch = [
        pltpu.VMEM((N,), f32),
        pltpu.VMEM((N,), f32),
        pltpu.VMEM((N,), f32),
        pltpu.VMEM((SEG * L,), f32),
        pltpu.VMEM((NSUP * L,), f32),
        pltpu.VMEM((REC,), f32),
        pltpu.VMEM((REC,), f32),
        pltpu.VMEM((REC,), f32),
        pltpu.VMEM((REC,), f32),
        pltpu.SemaphoreType.DMA,
        pltpu.SemaphoreType.DMA,
        pltpu.SemaphoreType.DMA,
    ]
    fn = pl.kernel(_sc_body,
                   out_type=[jax.ShapeDtypeStruct((B, REC), f32)],
                   mesh=mesh,
                   compiler_params=pltpu.CompilerParams(
                       needs_layout_passes=False),
                   scratch_types=scratch)
    return fn(s_sem, s_struct)


def _stats_body(x_ref, relid_ref, dirid_ref, rel_emb_ref, dir_emb_ref,
                w1_ref, b1_ref, out_ref, pre_ref):
    nf = jnp.float32(N)
    x = x_ref[...]
    m = jnp.max(x, axis=1, keepdims=True)
    mean = jnp.sum(x, axis=1, keepdims=True) / nf
    var = jnp.sum(x * x, axis=1, keepdims=True) / nf - mean * mean
    std = jnp.sqrt(jnp.maximum(var, 0.0))
    e = jnp.exp(x - m)
    s1 = jnp.sum(e, axis=1, keepdims=True)
    s2 = jnp.sum(e * x, axis=1, keepdims=True)
    ent = m + jnp.log(s1) - s2 / s1
    gap = m - mean
    z = jnp.zeros_like(mean)
    out_ref[...] = jnp.concatenate(
        [mean, std, m, gap, ent, z, z, z], axis=1)

    rel_oh = (relid_ref[...] ==
              lax.broadcasted_iota(jnp.int32, (RB, rel_emb_ref.shape[0]), 1)
              ).astype(jnp.float32)
    dir_oh = (dirid_ref[...] ==
              lax.broadcasted_iota(jnp.int32, (RB, 2), 1)).astype(jnp.float32)
    rel_vec = jnp.dot(rel_oh, rel_emb_ref[...],
                      preferred_element_type=jnp.float32)
    dir_vec = jnp.dot(dir_oh, dir_emb_ref[...],
                      preferred_element_type=jnp.float32)
    pre_ref[...] = (jnp.dot(rel_vec, w1_ref[6:38, :],
                            preferred_element_type=jnp.float32) +
                    jnp.dot(dir_vec, w1_ref[38:46, :],
                            preferred_element_type=jnp.float32) +
                    b1_ref[...])


def _stats_stage(s_sem, relid, dirid, rel_emb, dir_emb, W1, b1):
    hid = W1.shape[1]
    return pl.pallas_call(
        _stats_body,
        grid=(B // RB,),
        in_specs=[
            pl.BlockSpec((RB, N), lambda i: (i, 0)),
            pl.BlockSpec((RB, 1), lambda i: (i, 0)),
            pl.BlockSpec((RB, 1), lambda i: (i, 0)),
            pl.BlockSpec(rel_emb.shape, lambda i: (0, 0)),
            pl.BlockSpec(dir_emb.shape, lambda i: (0, 0)),
            pl.BlockSpec(W1.shape, lambda i: (0, 0)),
            pl.BlockSpec((1, hid), lambda i: (0, 0)),
        ],
        out_specs=[
            pl.BlockSpec((RB, 8), lambda i: (i, 0)),
            pl.BlockSpec((RB, hid), lambda i: (i, 0)),
        ],
        out_shape=[
            jax.ShapeDtypeStruct((B, 8), jnp.float32),
            jax.ShapeDtypeStruct((B, hid), jnp.float32),
        ],
    )(s_sem, relid, dirid, rel_emb, dir_emb, W1, b1.reshape(1, hid))


def _tc_body(stats_ref, comb_ref, pre_ref, w1_ref, w2_ref, b2_ref, inv_ref,
             out_ref):
    comb = comb_ref[...]
    stats = stats_ref[...]

    vA = comb[:, OFF_SVAL:OFF_SIDX]
    iA = comb[:, OFF_SIDX:OFF_TVAL]
    vB = comb[:, OFF_TVAL:OFF_TIDX]
    iB = comb[:, OFF_TIDX:REC]
    selA, selB = [], []
    for _ in range(TOPK):
        mxA = jnp.max(vA, axis=1, keepdims=True)
        mxB = jnp.max(vB, axis=1, keepdims=True)
        siA = jnp.min(jnp.where(vA == mxA, iA, jnp.float32(1e9)),
                      axis=1, keepdims=True)
        siB = jnp.min(jnp.where(vB == mxB, iB, jnp.float32(1e9)),
                      axis=1, keepdims=True)
        selA.append(siA)
        selB.append(siB)
        vA = jnp.where(iA == siA, -jnp.inf, vA)
        vB = jnp.where(iB == siB, -jnp.inf, vB)
    strsel = jnp.concatenate(selB, axis=1)

    match = jnp.zeros((B,), jnp.float32)
    for i in range(TOPK):
        hit = jnp.max((selA[i] == strsel).astype(jnp.float32), axis=1)
        match = match + hit
    agree = match * inv_ref[0, 0]

    x = jnp.concatenate([stats[:, 0:5], agree[:, None]], axis=1)
    h = jnp.maximum(jnp.dot(x, w1_ref[0:6, :],
                            preferred_element_type=jnp.float32) + pre_ref[...],
                    0.0)
    z = jnp.dot(h, w2_ref[...], preferred_element_type=jnp.float32) + b2_ref[...]
    r = 1.0 / (1.0 + jnp.exp(-z))
    out_ref[...] = jnp.clip(r, 0.05, 0.95)


def kernel(s_sem, s_struct, rel_ids, dir_ids, topm, rel_emb, dir_emb,
           W1, b1, W2, b2):
    (comb,) = _sc_stage(s_sem, s_struct)

    inv_topm = (1.0 / jnp.asarray(topm, jnp.float32)).reshape(1, 1)
    relid = rel_ids.astype(jnp.int32).reshape(B, 1)
    dirid = dir_ids.astype(jnp.int32).reshape(B, 1)
    stats, pre = _stats_stage(s_sem, relid, dirid, rel_emb, dir_emb, W1, b1)

    out = pl.pallas_call(
        _tc_body,
        out_shape=jax.ShapeDtypeStruct((B, 1), jnp.float32),
    )(stats, comb, pre, W1, W2, b2.reshape(1, 1), inv_topm)
    return out[:, 0]


# speedup vs baseline: 1.0668x; 1.0207x over previous
import functools

import jax
import jax.numpy as jnp
from jax import lax
from jax.experimental import pallas as pl
from jax.experimental.pallas import tpu as pltpu
from jax.experimental.pallas import tpu_sc as plsc

B = 128
N = 32768
L = 16
NCH = N // L
NC, NS = 2, 16
NW = NC * NS
RPW = B // NW
TOPK = 10
SCH = 16
SEG = NCH // SCH
SPS = 8
NSUP = SEG // SPS
RB = 8

OFF_SVAL = 0
OFF_SIDX = 160
OFF_TVAL = 320
OFF_TIDX = 480
REC = 640


def _merge_chain(va, ia, vb, ib):
    c = (vb > va) | ((vb == va) & (ib < ia))
    return jnp.where(c, vb, va), jnp.where(c, ib, ia)


def _pass1(buf, seg_val):
    ninf = jnp.full((L,), -jnp.inf, jnp.float32)

    def one_seg(base):
        sa, sb = ninf, ninf
        for j in range(SCH):
            x = buf[pl.ds(base + j * L, L)]
            if j % 2 == 0:
                sa = jnp.maximum(sa, x)
            else:
                sb = jnp.maximum(sb, x)
        return jnp.maximum(sa, sb)

    def body(g, _):
        for u in range(2):
            sg = g * 2 + u
            seg_val[pl.ds(sg * L, L)] = one_seg(sg * (SCH * L))
        return 0

    lax.fori_loop(0, SEG // 2, body, 0)


def _build_supseg(seg_val, supseg_val):
    def body(t, _):
        vs = [seg_val[pl.ds(t * (SPS * L) + j * L, L)] for j in range(SPS)]
        while len(vs) > 1:
            vs = [jnp.maximum(vs[i], vs[i + 1]) for i in range(0, len(vs), 2)]
        supseg_val[pl.ds(t * L, L)] = vs[0]
        return 0

    lax.fori_loop(0, NSUP, body, 0)


def _extract10(buf, seg_val, supseg_val, lane_i, stage, val_off, idx_off):
    ninf = jnp.full((L,), -jnp.inf, jnp.float32)
    zi = jnp.zeros((L,), jnp.int32)

    def body(k, _):
        bva, bta, bvb, btb = ninf, zi, ninf, zi
        for t in range(NSUP):
            v = supseg_val[pl.ds(t * L, L)]
            if t % 2 == 0:
                c = v > bva
                bva = jnp.where(c, v, bva)
                bta = jnp.where(c, zi + t, bta)
            else:
                c = v > bvb
                bvb = jnp.where(c, v, bvb)
                btb = jnp.where(c, zi + t, btb)
        bv, bt = _merge_chain(bva, bta, bvb, btb)
        dva, bsa, d2a, dvb, bsb, d2b = ninf, zi, ninf, ninf, zi, ninf
        for j in range(SPS):
            sj = bt * SPS + j
            g = plsc.load_gather(seg_val, [sj * L + lane_i])
            if j % 2 == 0:
                c = g > dva
                d2a = jnp.where(c, dva, jnp.maximum(d2a, g))
                dva = jnp.where(c, g, dva)
                bsa = jnp.where(c, sj, bsa)
            else:
                c = g > dvb
                d2b = jnp.where(c, dvb, jnp.maximum(d2b, g))
                dvb = jnp.where(c, g, dvb)
                bsb = jnp.where(c, sj, bsb)
        _, bs = _merge_chain(dva, bsa, dvb, bsb)
        d2 = jnp.maximum(jnp.minimum(dva, dvb), jnp.maximum(d2a, d2b))
        sbase = bs * (SCH * L) + lane_i
        nva, nia, n2a, nvb, nib, n2b = ninf, zi, ninf, ninf, zi, ninf
        for j in range(SCH):
            gidx = sbase + j * L
            g = plsc.load_gather(buf, [gidx])
            if j % 2 == 0:
                c = g > nva
                n2a = jnp.where(c, nva, jnp.maximum(n2a, g))
                nva = jnp.where(c, g, nva)
                nia = jnp.where(c, gidx, nia)
            else:
                c = g > nvb
                n2b = jnp.where(c, nvb, jnp.maximum(n2b, g))
                nvb = jnp.where(c, g, nvb)
                nib = jnp.where(c, gidx, nib)
        _, bi = _merge_chain(nva, nia, nvb, nib)
        m2 = jnp.maximum(jnp.minimum(nva, nvb), jnp.maximum(n2a, n2b))
        plsc.store_scatter(buf, [bi], ninf)
        stage[pl.ds(val_off + k * L, L)] = bv
        stage[pl.ds(idx_off + k * L, L)] = bi.astype(jnp.float32)
        plsc.store_scatter(seg_val, [bs * L + lane_i], m2)
        plsc.store_scatter(supseg_val, [bt * L + lane_i],
                           jnp.maximum(d2, m2))
        return 0

    lax.fori_loop(0, TOPK, body, 0)


def _topk_row(buf, seg_val, supseg_val, lane_i, stage, voff, ioff):
    _pass1(buf, seg_val)
    _build_supseg(seg_val, supseg_val)
    _extract10(buf, seg_val, supseg_val, lane_i, stage, voff, ioff)


def _sc_body(sem_hbm, struct_hbm, out_hbm,
             sem_a, sem_b, struct_v, seg_val, supseg_val,
             stage0, stage1, stage2, stage3, ds_sem, ds_str, ds_out):
    wid = lax.axis_index("s") * NC + lax.axis_index("c")
    lane_i = lax.broadcasted_iota(jnp.int32, (L,), 0)
    r0 = wid * RPW

    sem_bufs = [sem_a, sem_b]
    stage_bufs = [stage0, stage1, stage2, stage3]
    cp_sem = pltpu.async_copy(sem_hbm.at[r0], sem_a, ds_sem)
    out_cps = []
    for rr in range(RPW):
        r = r0 + rr
        cur = sem_bufs[rr % 2]
        stage = stage_bufs[rr]
        cp_struct = pltpu.async_copy(struct_hbm.at[r], struct_v, ds_str)
        cp_sem.wait()
        if rr + 1 < RPW:
            cp_sem = pltpu.async_copy(sem_hbm.at[r + 1],
                                      sem_bufs[(rr + 1) % 2], ds_sem)

        _topk_row(cur, seg_val, supseg_val, lane_i, stage,
                  OFF_SVAL, OFF_SIDX)
        cp_struct.wait()
        _topk_row(struct_v, seg_val, supseg_val, lane_i, stage,
                  OFF_TVAL, OFF_TIDX)

        out_cps.append(pltpu.async_copy(stage, out_hbm.at[r], ds_out))
    for cp in out_cps:
        cp.wait()


def _sc_stage(s_sem, s_struct):
    mesh = plsc.VectorSubcoreMesh(core_axis_name="c", subcore_axis_name="s",
                                  num_cores=NC, num_subcores=NS)
    f32 = jnp.float32
    scratch = [
        pltpu.VMEM((N,), f32),
        pltpu.VMEM((N,), f32),
        pltpu.VMEM((N,), f32),
        pltpu.VMEM((SEG * L,), f32),
        pltpu.VMEM((NSUP * L,), f32),
        pltpu.VMEM((REC,), f32),
        pltpu.VMEM((REC,), f32),
        pltpu.VMEM((REC,), f32),
        pltpu.VMEM((REC,), f32),
        pltpu.SemaphoreType.DMA,
        pltpu.SemaphoreType.DMA,
        pltpu.SemaphoreType.DMA,
    ]
    fn = pl.kernel(_sc_body,
                   out_type=[jax.ShapeDtypeStruct((B, REC), f32)],
                   mesh=mesh,
                   compiler_params=pltpu.CompilerParams(
                       needs_layout_passes=False),
                   scratch_types=scratch)
    return fn(s_sem, s_struct)


def _stats_body(x_ref, relid_ref, dirid_ref, rel_emb_ref, dir_emb_ref,
                w1_ref, b1_ref, out_ref, pre_ref):
    nf = jnp.float32(N)
    x = x_ref[...]
    m = jnp.max(x, axis=1, keepdims=True)
    mean = jnp.sum(x, axis=1, keepdims=True) / nf
    var = jnp.sum(x * x, axis=1, keepdims=True) / nf - mean * mean
    std = jnp.sqrt(jnp.maximum(var, 0.0))
    e = jnp.exp(x - m)
    s1 = jnp.sum(e, axis=1, keepdims=True)
    s2 = jnp.sum(e * x, axis=1, keepdims=True)
    ent = m + jnp.log(s1) - s2 / s1
    gap = m - mean
    z = jnp.zeros_like(mean)
    out_ref[...] = jnp.concatenate(
        [mean, std, m, gap, ent, z, z, z], axis=1)

    rel_oh = (relid_ref[...] ==
              lax.broadcasted_iota(jnp.int32, (RB, rel_emb_ref.shape[0]), 1)
              ).astype(jnp.float32)
    dir_oh = (dirid_ref[...] ==
              lax.broadcasted_iota(jnp.int32, (RB, 2), 1)).astype(jnp.float32)
    rel_vec = jnp.dot(rel_oh, rel_emb_ref[...],
                      preferred_element_type=jnp.float32)
    dir_vec = jnp.dot(dir_oh, dir_emb_ref[...],
                      preferred_element_type=jnp.float32)
    pre_ref[...] = (jnp.dot(rel_vec, w1_ref[6:38, :],
                            preferred_element_type=jnp.float32) +
                    jnp.dot(dir_vec, w1_ref[38:46, :],
                            preferred_element_type=jnp.float32) +
                    b1_ref[...])


def _stats_stage(s_sem, relid, dirid, rel_emb, dir_emb, W1, b1):
    hid = W1.shape[1]
    return pl.pallas_call(
        _stats_body,
        grid=(B // RB,),
        in_specs=[
            pl.BlockSpec((RB, N), lambda i: (i, 0)),
            pl.BlockSpec((RB, 1), lambda i: (i, 0)),
            pl.BlockSpec((RB, 1), lambda i: (i, 0)),
            pl.BlockSpec(rel_emb.shape, lambda i: (0, 0)),
            pl.BlockSpec(dir_emb.shape, lambda i: (0, 0)),
            pl.BlockSpec(W1.shape, lambda i: (0, 0)),
            pl.BlockSpec((1, hid), lambda i: (0, 0)),
        ],
        out_specs=[
            pl.BlockSpec((RB, 8), lambda i: (i, 0)),
            pl.BlockSpec((RB, hid), lambda i: (i, 0)),
        ],
        out_shape=[
            jax.ShapeDtypeStruct((B, 8), jnp.float32),
            jax.ShapeDtypeStruct((B, hid), jnp.float32),
        ],
    )(s_sem, relid, dirid, rel_emb, dir_emb, W1, b1.reshape(1, hid))


def _tc_body(stats_ref, comb_ref, pre_ref, w1_ref, w2_ref, b2_ref, inv_ref,
             out_ref):
    comb = comb_ref[...]
    stats = stats_ref[...]

    vA = comb[:, OFF_SVAL:OFF_SIDX]
    iA = comb[:, OFF_SIDX:OFF_TVAL]
    vB = comb[:, OFF_TVAL:OFF_TIDX]
    iB = comb[:, OFF_TIDX:REC]
    selA, selB = [], []
    for _ in range(TOPK):
        mxA = jnp.max(vA, axis=1, keepdims=True)
        mxB = jnp.max(vB, axis=1, keepdims=True)
        siA = jnp.min(jnp.where(vA == mxA, iA, jnp.float32(1e9)),
                      axis=1, keepdims=True)
        siB = jnp.min(jnp.where(vB == mxB, iB, jnp.float32(1e9)),
                      axis=1, keepdims=True)
        selA.append(siA)
        selB.append(siB)
        vA = jnp.where(iA == siA, -jnp.inf, vA)
        vB = jnp.where(iB == siB, -jnp.inf, vB)
    strsel = jnp.concatenate(selB, axis=1)

    match = jnp.zeros((B,), jnp.float32)
    for i in range(TOPK):
        hit = jnp.max((selA[i] == strsel).astype(jnp.float32), axis=1)
        match = match + hit
    agree = match * inv_ref[0, 0]

    x = jnp.concatenate([stats[:, 0:5], agree[:, None]], axis=1)
    h = jnp.maximum(jnp.dot(x, w1_ref[0:6, :],
                            preferred_element_type=jnp.float32) + pre_ref[...],
                    0.0)
    z = jnp.dot(h, w2_ref[...], preferred_element_type=jnp.float32) + b2_ref[...]
    r = 1.0 / (1.0 + jnp.exp(-z))
    out_ref[...] = jnp.clip(r, 0.05, 0.95)


def kernel(s_sem, s_struct, rel_ids, dir_ids, topm, rel_emb, dir_emb,
           W1, b1, W2, b2):
    (comb,) = _sc_stage(s_sem, s_struct)

    inv_topm = (1.0 / jnp.asarray(topm, jnp.float32)).reshape(1, 1)
    relid = rel_ids.astype(jnp.int32).reshape(B, 1)
    dirid = dir_ids.astype(jnp.int32).reshape(B, 1)
    stats, pre = _stats_stage(s_sem, relid, dirid, rel_emb, dir_emb, W1, b1)

    out = pl.pallas_call(
        _tc_body,
        out_shape=jax.ShapeDtypeStruct((B, 1), jnp.float32),
    )(stats, comb, pre, W1, W2, b2.reshape(1, 1), inv_topm)
    return out[:, 0]


# speedup vs baseline: 1.0690x; 1.0021x over previous
import jax
import jax.numpy as jnp
from jax import lax
from jax.experimental import pallas as pl
from jax.experimental.pallas import tpu as pltpu
from jax.experimental.pallas import tpu_sc as plsc

B = 128
N = 32768
L = 16
NCH = N // L
NC, NS = 2, 16
NW = NC * NS
RPW = B // NW
TOPK = 10
SCH = 16
SEG = NCH // SCH
SPS = 8
NSUP = SEG // SPS
RB = 8

OFF_SVAL = 0
OFF_SIDX = 160
OFF_TVAL = 320
OFF_TIDX = 480
REC = 640


def _merge_chain(va, ia, vb, ib):
    c = (vb > va) | ((vb == va) & (ib < ia))
    return jnp.where(c, vb, va), jnp.where(c, ib, ia)


def _pass1(buf, seg_val):
    ninf = jnp.full((L,), -jnp.inf, jnp.float32)

    def one_seg(base):
        sa, sb = ninf, ninf
        for j in range(SCH):
            x = buf[pl.ds(base + j * L, L)]
            if j % 2 == 0:
                sa = jnp.maximum(sa, x)
            else:
                sb = jnp.maximum(sb, x)
        return jnp.maximum(sa, sb)

    def body(g, _):
        for u in range(2):
            sg = g * 2 + u
            seg_val[pl.ds(sg * L, L)] = one_seg(sg * (SCH * L))
        return 0

    lax.fori_loop(0, SEG // 2, body, 0)


def _build_supseg(seg_val, supseg_val):
    def body(t, _):
        vs = [seg_val[pl.ds(t * (SPS * L) + j * L, L)] for j in range(SPS)]
        while len(vs) > 1:
            vs = [jnp.maximum(vs[i], vs[i + 1]) for i in range(0, len(vs), 2)]
        supseg_val[pl.ds(t * L, L)] = vs[0]
        return 0

    lax.fori_loop(0, NSUP, body, 0)


def _extract10(buf, seg_val, supseg_val, lane_i, stage, val_off, idx_off):
    ninf = jnp.full((L,), -jnp.inf, jnp.float32)
    zi = jnp.zeros((L,), jnp.int32)

    def body(k, _):
        bva, bta, bvb, btb = ninf, zi, ninf, zi
        for t in range(NSUP):
            v = supseg_val[pl.ds(t * L, L)]
            if t % 2 == 0:
                c = v > bva
                bva = jnp.where(c, v, bva)
                bta = jnp.where(c, zi + t, bta)
            else:
                c = v > bvb
                bvb = jnp.where(c, v, bvb)
                btb = jnp.where(c, zi + t, btb)
        bv, bt = _merge_chain(bva, bta, bvb, btb)
        dva, bsa, d2a, dvb, bsb, d2b = ninf, zi, ninf, ninf, zi, ninf
        for j in range(SPS):
            sj = bt * SPS + j
            g = plsc.load_gather(seg_val, [sj * L + lane_i])
            if j % 2 == 0:
                c = g > dva
                d2a = jnp.where(c, dva, jnp.maximum(d2a, g))
                dva = jnp.where(c, g, dva)
                bsa = jnp.where(c, sj, bsa)
            else:
                c = g > dvb
                d2b = jnp.where(c, dvb, jnp.maximum(d2b, g))
                dvb = jnp.where(c, g, dvb)
                bsb = jnp.where(c, sj, bsb)
        _, bs = _merge_chain(dva, bsa, dvb, bsb)
        d2 = jnp.maximum(jnp.minimum(dva, dvb), jnp.maximum(d2a, d2b))
        sbase = bs * (SCH * L) + lane_i
        nva, nia, n2a, nvb, nib, n2b = ninf, zi, ninf, ninf, zi, ninf
        for j in range(SCH):
            gidx = sbase + j * L
            g = plsc.load_gather(buf, [gidx])
            if j % 2 == 0:
                c = g > nva
                n2a = jnp.where(c, nva, jnp.maximum(n2a, g))
                nva = jnp.where(c, g, nva)
                nia = jnp.where(c, gidx, nia)
            else:
                c = g > nvb
                n2b = jnp.where(c, nvb, jnp.maximum(n2b, g))
                nvb = jnp.where(c, g, nvb)
                nib = jnp.where(c, gidx, nib)
        _, bi = _merge_chain(nva, nia, nvb, nib)
        m2 = jnp.maximum(jnp.minimum(nva, nvb), jnp.maximum(n2a, n2b))
        plsc.store_scatter(buf, [bi], ninf)
        stage[pl.ds(val_off + k * L, L)] = bv
        stage[pl.ds(idx_off + k * L, L)] = bi.astype(jnp.float32)
        plsc.store_scatter(seg_val, [bs * L + lane_i], m2)
        plsc.store_scatter(supseg_val, [bt * L + lane_i],
                           jnp.maximum(d2, m2))
        return 0

    lax.fori_loop(0, TOPK, body, 0)


def _topk_row(buf, seg_val, supseg_val, lane_i, stage, voff, ioff):
    _pass1(buf, seg_val)
    _build_supseg(seg_val, supseg_val)
    _extract10(buf, seg_val, supseg_val, lane_i, stage, voff, ioff)


def _sc_body(sem_hbm, struct_hbm, out_hbm,
             sem_a, sem_b, struct_v, seg_val, supseg_val,
             stage0, stage1, stage2, stage3, ds_sem, ds_str, ds_out):
    wid = lax.axis_index("s") * NC + lax.axis_index("c")
    lane_i = lax.broadcasted_iota(jnp.int32, (L,), 0)
    r0 = wid * RPW

    sem_bufs = [sem_a, sem_b]
    stage_bufs = [stage0, stage1, stage2, stage3]
    cp_sem = pltpu.async_copy(sem_hbm.at[r0], sem_a, ds_sem)
    out_cps = []
    for rr in range(RPW):
        r = r0 + rr
        cur = sem_bufs[rr % 2]
        stage = stage_bufs[rr]
        cp_struct = pltpu.async_copy(struct_hbm.at[r], struct_v, ds_str)
        cp_sem.wait()
        if rr + 1 < RPW:
            cp_sem = pltpu.async_copy(sem_hbm.at[r + 1],
                                      sem_bufs[(rr + 1) % 2], ds_sem)

        _topk_row(cur, seg_val, supseg_val, lane_i, stage,
                  OFF_SVAL, OFF_SIDX)
        cp_struct.wait()
        _topk_row(struct_v, seg_val, supseg_val, lane_i, stage,
                  OFF_TVAL, OFF_TIDX)

        out_cps.append(pltpu.async_copy(stage, out_hbm.at[r], ds_out))
    for cp in out_cps:
        cp.wait()


def _sc_stage(s_sem, s_struct):
    mesh = plsc.VectorSubcoreMesh(core_axis_name="c", subcore_axis_name="s",
                                  num_cores=NC, num_subcores=NS)
    f32 = jnp.float32
    scratch = [
        pltpu.VMEM((N,), f32),
        pltpu.VMEM((N,), f32),
        pltpu.VMEM((N,), f32),
        pltpu.VMEM((SEG * L,), f32),
        pltpu.VMEM((NSUP * L,), f32),
        pltpu.VMEM((REC,), f32),
        pltpu.VMEM((REC,), f32),
        pltpu.VMEM((REC,), f32),
        pltpu.VMEM((REC,), f32),
        pltpu.SemaphoreType.DMA,
        pltpu.SemaphoreType.DMA,
        pltpu.SemaphoreType.DMA,
    ]
    fn = pl.kernel(_sc_body,
                   out_type=[jax.ShapeDtypeStruct((B, REC), f32)],
                   mesh=mesh,
                   compiler_params=pltpu.CompilerParams(
                       needs_layout_passes=False),
                   scratch_types=scratch)
    return fn(s_sem, s_struct)


def _stats_body(x_ref, relid_ref, dirid_ref, rel_emb_ref, dir_emb_ref,
                w1_ref, b1_ref, out_ref, pre_ref):
    nf = jnp.float32(N)
    x = x_ref[...]
    m = jnp.max(x, axis=1, keepdims=True)
    mean = jnp.sum(x, axis=1, keepdims=True) / nf
    var = jnp.sum(x * x, axis=1, keepdims=True) / nf - mean * mean
    std = jnp.sqrt(jnp.maximum(var, 0.0))
    e = jnp.exp(x - m)
    s1 = jnp.sum(e, axis=1, keepdims=True)
    s2 = jnp.sum(e * x, axis=1, keepdims=True)
    ent = m + jnp.log(s1) - s2 / s1
    gap = m - mean
    z = jnp.zeros_like(mean)
    out_ref[...] = jnp.concatenate(
        [mean, std, m, gap, ent, z, z, z], axis=1)

    rel_oh = (relid_ref[...] ==
              lax.broadcasted_iota(jnp.int32, (RB, rel_emb_ref.shape[0]), 1)
              ).astype(jnp.float32)
    dir_oh = (dirid_ref[...] ==
              lax.broadcasted_iota(jnp.int32, (RB, 2), 1)).astype(jnp.float32)
    rel_vec = jnp.dot(rel_oh, rel_emb_ref[...],
                      preferred_element_type=jnp.float32)
    dir_vec = jnp.dot(dir_oh, dir_emb_ref[...],
                      preferred_element_type=jnp.float32)
    pre_ref[...] = (jnp.dot(rel_vec, w1_ref[6:38, :],
                            preferred_element_type=jnp.float32) +
                    jnp.dot(dir_vec, w1_ref[38:46, :],
                            preferred_element_type=jnp.float32) +
                    b1_ref[...])


def _stats_stage(s_sem, relid, dirid, rel_emb, dir_emb, W1, b1):
    hid = W1.shape[1]
    return pl.pallas_call(
        _stats_body,
        grid=(B // RB,),
        in_specs=[
            pl.BlockSpec((RB, N), lambda i: (i, 0)),
            pl.BlockSpec((RB, 1), lambda i: (i, 0)),
            pl.BlockSpec((RB, 1), lambda i: (i, 0)),
            pl.BlockSpec(rel_emb.shape, lambda i: (0, 0)),
            pl.BlockSpec(dir_emb.shape, lambda i: (0, 0)),
            pl.BlockSpec(W1.shape, lambda i: (0, 0)),
            pl.BlockSpec((1, hid), lambda i: (0, 0)),
        ],
        out_specs=[
            pl.BlockSpec((RB, 8), lambda i: (i, 0)),
            pl.BlockSpec((RB, hid), lambda i: (i, 0)),
        ],
        out_shape=[
            jax.ShapeDtypeStruct((B, 8), jnp.float32),
            jax.ShapeDtypeStruct((B, hid), jnp.float32),
        ],
    )(s_sem, relid, dirid, rel_emb, dir_emb, W1, b1.reshape(1, hid))


def _tc_body(stats_ref, comb_ref, pre_ref, w1_ref, w2_ref, b2_ref, inv_ref,
             out_ref):
    comb = comb_ref[...]
    stats = stats_ref[...]

    vA = comb[:, OFF_SVAL:OFF_SIDX]
    iA = comb[:, OFF_SIDX:OFF_TVAL]
    vB = comb[:, OFF_TVAL:OFF_TIDX]
    iB = comb[:, OFF_TIDX:REC]
    selA, selB = [], []
    for _ in range(TOPK):
        mxA = jnp.max(vA, axis=1, keepdims=True)
        mxB = jnp.max(vB, axis=1, keepdims=True)
        siA = jnp.min(jnp.where(vA == mxA, iA, jnp.float32(1e9)),
                      axis=1, keepdims=True)
        siB = jnp.min(jnp.where(vB == mxB, iB, jnp.float32(1e9)),
                      axis=1, keepdims=True)
        selA.append(siA)
        selB.append(siB)
        vA = jnp.where(iA == siA, -jnp.inf, vA)
        vB = jnp.where(iB == siB, -jnp.inf, vB)
    strsel = jnp.concatenate(selB, axis=1)

    match = jnp.zeros((B,), jnp.float32)
    for i in range(TOPK):
        hit = jnp.max((selA[i] == strsel).astype(jnp.float32), axis=1)
        match = match + hit
    agree = match * inv_ref[0, 0]

    x = jnp.concatenate([stats[:, 0:5], agree[:, None]], axis=1)
    h = jnp.maximum(jnp.dot(x, w1_ref[0:6, :],
                            preferred_element_type=jnp.float32) + pre_ref[...],
                    0.0)
    z = jnp.dot(h, w2_ref[...], preferred_element_type=jnp.float32) + b2_ref[...]
    r = 1.0 / (1.0 + jnp.exp(-z))
    out_ref[...] = jnp.clip(r, 0.05, 0.95)


def kernel(s_sem, s_struct, rel_ids, dir_ids, topm, rel_emb, dir_emb,
           W1, b1, W2, b2):
    (comb,) = _sc_stage(s_sem, s_struct)

    inv_topm = (1.0 / jnp.asarray(topm, jnp.float32)).reshape(1, 1)
    relid = rel_ids.astype(jnp.int32).reshape(B, 1)
    dirid = dir_ids.astype(jnp.int32).reshape(B, 1)
    stats, pre = _stats_stage(s_sem, relid, dirid, rel_emb, dir_emb, W1, b1)

    out = pl.pallas_call(
        _tc_body,
        out_shape=jax.ShapeDtypeStruct((B, 1), jnp.float32),
    )(stats, comb, pre, W1, W2, b2.reshape(1, 1), inv_topm)
    return out[:, 0]
